# baseline (speedup 1.0000x reference)
import functools

import jax
import jax.numpy as jnp
from jax.experimental import pallas as pl

NJ = 17
SUBSETS = [[0, 5, 6, 11, 12], [7, 8, 13, 14], [9, 10, 15, 16]]
_ms = [frozenset(s) for s in SUBSETS]
NB = [sorted(_ms[0] | _ms[1]), sorted(_ms[0] | _ms[1] | _ms[2]), sorted(_ms[1] | _ms[2])]
MASKS = [sorted(s) for s in _ms]


def _body(kxy_ref, sc_ref, w40_ref, w41_ref, w42_ref, we_ref, be_ref,
          wp_ref, bp_ref, out_ref):
    kxyt = jnp.transpose(kxy_ref[...], (1, 0))
    sct = jnp.transpose(sc_ref[...], (1, 0))
    bn = kxyt.shape[1]
    row = jax.lax.broadcasted_iota(jnp.int32, (2 * NJ, bn), 0)
    is_x = (row % 2) == 0
    big = jnp.float32(1e30)
    mnx = jnp.min(jnp.where(is_x, kxyt, big), axis=0, keepdims=True)
    mny = jnp.min(jnp.where(is_x, big, kxyt), axis=0, keepdims=True)
    mxx = jnp.max(jnp.where(is_x, kxyt, -big), axis=0, keepdims=True)
    mxy = jnp.max(jnp.where(is_x, -big, kxyt), axis=0, keepdims=True)
    mn = jnp.where(is_x, mnx, mny)
    inv = jnp.where(is_x, 1.0 / (mxx - mnx + 1e-6), 1.0 / (mxy - mny + 1e-6))
    nt = (kxyt - mn) * inv

    m = NJ * bn
    xrow = jnp.concatenate([nt[2 * j:2 * j + 1, :] for j in range(NJ)], axis=1)
    yrow = jnp.concatenate([nt[2 * j + 1:2 * j + 2, :] for j in range(NJ)], axis=1)
    srow = jnp.concatenate([sct[j:j + 1, :] for j in range(NJ)], axis=1)
    onerow = jnp.ones((1, m), dtype=jnp.float32)
    p4 = jnp.concatenate([xrow, yrow, srow, onerow], axis=0)

    we = we_ref[...]
    we2 = we[:, 64:128]
    ab = jnp.concatenate([we[:, 0:64] - we2, be_ref[...]], axis=1)
    wp = wp_ref[...]
    w4s = [w40_ref, w41_ref, w42_ref]

    acc = jnp.broadcast_to(bp_ref[...], (128, bn))
    for lvl in range(3):
        h = jnp.dot(w4s[lvl][...], p4, preferred_element_type=jnp.float32)
        fm = jnp.maximum(h, 0.0) * srow
        csl = lambda arr, j: arr[:, j * bn:(j + 1) * bn]
        ssum = functools.reduce(jnp.add, [csl(fm, j) for j in MASKS[lvl]])
        f5 = jnp.concatenate([fm, onerow], axis=0)
        ct = jnp.dot(we2, fm, preferred_element_type=jnp.float32)
        at = jnp.dot(ab, f5, preferred_element_type=jnp.float32)
        ks = NB[lvl]
        c = {k: csl(ct, k) for k in ks}
        m1 = jnp.maximum(c[ks[0]], c[ks[1]])
        m2 = jnp.minimum(c[ks[0]], c[ks[1]])
        for k in ks[2:]:
            m2 = jnp.maximum(m2, jnp.minimum(m1, c[k]))
            m1 = jnp.maximum(m1, c[k])
        zsum = functools.reduce(jnp.add, [
            jnp.maximum(csl(at, j) + jnp.where(c[j] == m1, m2, m1), 0.0)
            for j in MASKS[lvl]])
        inv_cnt = 1.0 / float(len(MASKS[lvl]))
        acc = acc + jnp.dot(wp[:, 128 * lvl:128 * lvl + 64], ssum,
                            preferred_element_type=jnp.float32)
        acc = acc + jnp.dot(wp[:, 128 * lvl + 64:128 * lvl + 128],
                            zsum * inv_cnt,
                            preferred_element_type=jnp.float32)
    out_ref[...] = jnp.transpose(acc, (1, 0))


def kernel(keypoints, scores, W0, b0, W1, b1, W2, b2, We, be, Wp, bp):
    n = keypoints.shape[0]
    block_n = 256
    kxy = keypoints.reshape(n, 2 * NJ)
    w4 = [jnp.concatenate([w, b[:, None]], axis=1)
          for w, b in ((W0, b0), (W1, b1), (W2, b2))]
    dspec = lambda c: pl.BlockSpec((block_n, c), lambda i: (i, 0))

    def rep(shape):
        return pl.BlockSpec(shape, lambda i: tuple(0 for _ in shape))

    return pl.pallas_call(
        _body,
        grid=(n // block_n,),
        in_specs=[
            dspec(2 * NJ), dspec(NJ),
            rep((64, 4)), rep((64, 4)), rep((64, 4)),
            rep((64, 128)), rep((64, 1)),
            rep((128, 384)), rep((128, 1)),
        ],
        out_specs=pl.BlockSpec((block_n, 128), lambda i: (i, 0)),
        out_shape=jax.ShapeDtypeStruct((n, 128), jnp.float32),
    )(kxy, scores, w4[0], w4[1], w4[2], We, be[:, None], Wp, bp[:, None])


# speedup vs baseline: 1.2341x; 1.2341x over previous
import functools

import jax
import jax.numpy as jnp
from jax.experimental import pallas as pl

NJ = 17
SUBSETS = [[0, 5, 6, 11, 12], [7, 8, 13, 14], [9, 10, 15, 16]]
_ms = [frozenset(s) for s in SUBSETS]
NB = [sorted(_ms[0] | _ms[1]), sorted(_ms[0] | _ms[1] | _ms[2]), sorted(_ms[1] | _ms[2])]
MASKS = [sorted(s) for s in _ms]


def _body(kxyt_ref, sct_ref, w40_ref, w41_ref, w42_ref, we_ref, be_ref,
          wp_ref, bp_ref, out_ref):
    kxyt = kxyt_ref[...]
    sct = sct_ref[...]
    bn = kxyt.shape[1]
    row = jax.lax.broadcasted_iota(jnp.int32, (2 * NJ, bn), 0)
    is_x = (row % 2) == 0
    big = jnp.float32(1e30)
    mnx = jnp.min(jnp.where(is_x, kxyt, big), axis=0, keepdims=True)
    mny = jnp.min(jnp.where(is_x, big, kxyt), axis=0, keepdims=True)
    mxx = jnp.max(jnp.where(is_x, kxyt, -big), axis=0, keepdims=True)
    mxy = jnp.max(jnp.where(is_x, -big, kxyt), axis=0, keepdims=True)
    mn = jnp.where(is_x, mnx, mny)
    inv = jnp.where(is_x, 1.0 / (mxx - mnx + 1e-6), 1.0 / (mxy - mny + 1e-6))
    nt = (kxyt - mn) * inv

    m = NJ * bn
    xrow = jnp.concatenate([nt[2 * j:2 * j + 1, :] for j in range(NJ)], axis=1)
    yrow = jnp.concatenate([nt[2 * j + 1:2 * j + 2, :] for j in range(NJ)], axis=1)
    srow = jnp.concatenate([sct[j:j + 1, :] for j in range(NJ)], axis=1)
    onerow = jnp.ones((1, m), dtype=jnp.float32)
    p4 = jnp.concatenate([xrow, yrow, srow, onerow], axis=0)

    we = we_ref[...]
    we2 = we[:, 64:128]
    ab = jnp.concatenate([we[:, 0:64] - we2, be_ref[...]], axis=1)
    wp = wp_ref[...]
    w4s = [w40_ref, w41_ref, w42_ref]

    acc = jnp.broadcast_to(bp_ref[...], (128, bn))
    for lvl in range(3):
        h = jnp.dot(w4s[lvl][...], p4, preferred_element_type=jnp.float32)
        fm = jnp.maximum(h, 0.0) * srow
        csl = lambda arr, j: arr[:, j * bn:(j + 1) * bn]
        ssum = functools.reduce(jnp.add, [csl(fm, j) for j in MASKS[lvl]])
        f5 = jnp.concatenate([fm, onerow], axis=0)
        ct = jnp.dot(we2, fm, preferred_element_type=jnp.float32)
        at = jnp.dot(ab, f5, preferred_element_type=jnp.float32)
        ks = NB[lvl]
        c = {k: csl(ct, k) for k in ks}
        m1 = jnp.maximum(c[ks[0]], c[ks[1]])
        m2 = jnp.minimum(c[ks[0]], c[ks[1]])
        for k in ks[2:]:
            m2 = jnp.maximum(m2, jnp.minimum(m1, c[k]))
            m1 = jnp.maximum(m1, c[k])
        zsum = functools.reduce(jnp.add, [
            jnp.maximum(csl(at, j) + jnp.where(c[j] == m1, m2, m1), 0.0)
            for j in MASKS[lvl]])
        inv_cnt = 1.0 / float(len(MASKS[lvl]))
        acc = acc + jnp.dot(wp[:, 128 * lvl:128 * lvl + 64], ssum,
                            preferred_element_type=jnp.float32)
        acc = acc + jnp.dot(wp[:, 128 * lvl + 64:128 * lvl + 128],
                            zsum * inv_cnt,
                            preferred_element_type=jnp.float32)
    out_ref[...] = jnp.transpose(acc, (1, 0))


def kernel(keypoints, scores, W0, b0, W1, b1, W2, b2, We, be, Wp, bp):
    n = keypoints.shape[0]
    block_n = 512
    kxyt = keypoints.reshape(n, 2 * NJ).T
    sct = scores.T
    w4 = [jnp.concatenate([w, b[:, None]], axis=1)
          for w, b in ((W0, b0), (W1, b1), (W2, b2))]
    tspec = lambda r: pl.BlockSpec((r, block_n), lambda i: (0, i))

    def rep(shape):
        return pl.BlockSpec(shape, lambda i: tuple(0 for _ in shape))

    return pl.pallas_call(
        _body,
        grid=(n // block_n,),
        in_specs=[
            tspec(2 * NJ), tspec(NJ),
            rep((64, 4)), rep((64, 4)), rep((64, 4)),
            rep((64, 128)), rep((64, 1)),
            rep((128, 384)), rep((128, 1)),
        ],
        out_specs=pl.BlockSpec((block_n, 128), lambda i: (i, 0)),
        out_shape=jax.ShapeDtypeStruct((n, 128), jnp.float32),
    )(kxyt, sct, w4[0], w4[1], w4[2], We, be[:, None], Wp, bp[:, None])
